# baseline (speedup 1.0000x reference)
import functools

import jax
import jax.numpy as jnp
from jax import lax
from jax.experimental import pallas as pl
from jax.experimental.pallas import tpu as pltpu
from jax.experimental.pallas import tpu_sc as plsc

VOCAB = 1000000
DIM = 64
MAX_LEN = 200
BATCH = 4096
SCALE = 8.0

NC = 2
NS = 16
NW = NC * NS
ROWS_W = BATCH * MAX_LEN // NW
CHUNKS = ROWS_W // MAX_LEN


def _positional_encoding():
    depth = DIM // 2
    positions = jnp.arange(MAX_LEN)[:, None]
    depths = jnp.arange(depth)[None, :] / depth
    angle_rates = 1.0 / (10000.0 ** depths)
    angle_rads = positions * angle_rates
    pos = jnp.concatenate([jnp.sin(angle_rads), jnp.cos(angle_rads)], axis=-1)
    return pos.astype(jnp.float32)


def _tpe_body(x_hbm, pos_hbm, table_hbm, out_hbm, idx_v, pos_v, rows_v, sem):
    wid = lax.axis_index("s") * NC + lax.axis_index("c")
    base_w = wid * ROWS_W
    pltpu.sync_copy(pos_hbm, pos_v)

    def chunk_body(c, carry):
        base = base_w + c * MAX_LEN
        pltpu.sync_copy(x_hbm.at[pl.ds(base, MAX_LEN)], idx_v)
        d1 = pltpu.async_copy(
            table_hbm.at[idx_v.at[pl.ds(0, 128)]], rows_v.at[pl.ds(0, 128)], sem)
        d2 = pltpu.async_copy(
            table_hbm.at[idx_v.at[pl.ds(128, 72)]], rows_v.at[pl.ds(128, 72)], sem)
        d1.wait()
        d2.wait()

        def row_body(r, rcarry):
            for j in range(DIM // 16):
                sl = pl.ds(j * 16, 16)
                rows_v[r, sl] = rows_v[r, sl] * SCALE + pos_v[r, sl]
            return rcarry
        lax.fori_loop(0, MAX_LEN, row_body, 0, unroll=8)

        pltpu.sync_copy(rows_v, out_hbm.at[pl.ds(base, MAX_LEN)])
        return carry

    lax.fori_loop(0, CHUNKS, chunk_body, 0)


@jax.jit
def kernel(x, token_table):
    pos = _positional_encoding()
    xf = x.reshape(BATCH * MAX_LEN)
    mesh = plsc.VectorSubcoreMesh(core_axis_name="c", subcore_axis_name="s")
    run = pl.kernel(
        _tpe_body,
        out_type=jax.ShapeDtypeStruct((BATCH * MAX_LEN, DIM), jnp.float32),
        mesh=mesh,
        scratch_types=[
            pltpu.VMEM((MAX_LEN,), jnp.int32),
            pltpu.VMEM((MAX_LEN, DIM), jnp.float32),
            pltpu.VMEM((MAX_LEN, DIM), jnp.float32),
            pltpu.SemaphoreType.DMA,
        ],
        compiler_params=pltpu.CompilerParams(use_tc_tiling_on_sc=False),
    )
    outf = run(xf, pos, token_table)
    return outf.reshape(BATCH, MAX_LEN, DIM)


# speedup vs baseline: 1.2979x; 1.2979x over previous
import jax
import jax.numpy as jnp
from jax import lax
from jax.experimental import pallas as pl
from jax.experimental.pallas import tpu as pltpu
from jax.experimental.pallas import tpu_sc as plsc

VOCAB = 1000000
DIM = 64
MAX_LEN = 200
BATCH = 4096
SCALE = 8.0

NC = 2
NS = 16
NW = NC * NS
ROWS_W = BATCH * MAX_LEN // NW
CHUNK = 2 * MAX_LEN
NCHUNK = ROWS_W // CHUNK
NPAIR = NCHUNK // 2
NG = CHUNK // 128
GR = CHUNK - NG * 128


def _positional_encoding():
    depth = DIM // 2
    positions = jnp.arange(MAX_LEN)[:, None]
    depths = jnp.arange(depth)[None, :] / depth
    angle_rates = 1.0 / (10000.0 ** depths)
    angle_rads = positions * angle_rates
    pos = jnp.concatenate([jnp.sin(angle_rads), jnp.cos(angle_rads)], axis=-1)
    return pos.astype(jnp.float32)


def _gather_descs(table_hbm, idx_all, base, buf, sem):
    descs = []
    for k in range(NG):
        descs.append(pltpu.make_async_copy(
            table_hbm.at[idx_all.at[pl.ds(base + k * 128, 128)]],
            buf.at[pl.ds(k * 128, 128)], sem))
    if GR:
        descs.append(pltpu.make_async_copy(
            table_hbm.at[idx_all.at[pl.ds(base + NG * 128, GR)]],
            buf.at[pl.ds(NG * 128, GR)], sem))
    return descs


def _fire_gathers(table_hbm, idx_all, base, buf, sem):
    for d in _gather_descs(table_hbm, idx_all, base, buf, sem):
        d.start()


def _wait_gathers(table_hbm, idx_all, base, buf, sem):
    for d in _gather_descs(table_hbm, idx_all, base, buf, sem):
        d.wait()


def _tpe_body(x_hbm, pos_hbm, table_hbm, out_hbm,
              idx_all, pos_v, buf_a, buf_b,
              sem_ga, sem_gb, sem_wa, sem_wb):
    wid = lax.axis_index("s") * NC + lax.axis_index("c")
    base_w = wid * ROWS_W
    pltpu.sync_copy(x_hbm.at[pl.ds(base_w, ROWS_W)], idx_all)
    pltpu.sync_copy(pos_hbm, pos_v)

    def compute(buf):
        def row_body(r, carry):
            for j in range(DIM // 16):
                sl = pl.ds(j * 16, 16)
                pv = pos_v[r, sl]
                buf[r, sl] = buf[r, sl] * SCALE + pv
                buf[r + MAX_LEN, sl] = buf[r + MAX_LEN, sl] * SCALE + pv
            return carry
        lax.fori_loop(0, MAX_LEN, row_body, 0, unroll=4)

    _fire_gathers(table_hbm, idx_all, 0, buf_a, sem_ga)

    def pair_body(c2, carry):
        base_a = (2 * c2) * CHUNK
        base_b = (2 * c2 + 1) * CHUNK

        @pl.when(c2 > 0)
        def _():
            pltpu.make_async_copy(
                buf_b, out_hbm.at[pl.ds(base_w + base_b - 2 * CHUNK, CHUNK)],
                sem_wb).wait()
        _fire_gathers(table_hbm, idx_all, base_b, buf_b, sem_gb)

        _wait_gathers(table_hbm, idx_all, base_a, buf_a, sem_ga)
        compute(buf_a)
        pltpu.make_async_copy(
            buf_a, out_hbm.at[pl.ds(base_w + base_a, CHUNK)], sem_wa).start()

        _wait_gathers(table_hbm, idx_all, base_b, buf_b, sem_gb)
        compute(buf_b)
        pltpu.make_async_copy(
            buf_b, out_hbm.at[pl.ds(base_w + base_b, CHUNK)], sem_wb).start()

        pltpu.make_async_copy(
            buf_a, out_hbm.at[pl.ds(base_w + base_a, CHUNK)], sem_wa).wait()

        @pl.when(c2 + 1 < NPAIR)
        def _():
            _fire_gathers(table_hbm, idx_all, base_a + 2 * CHUNK, buf_a, sem_ga)
        return carry

    lax.fori_loop(0, NPAIR, pair_body, 0)

    pltpu.make_async_copy(
        buf_b, out_hbm.at[pl.ds(base_w + ROWS_W - CHUNK, CHUNK)], sem_wb).wait()


@jax.jit
def kernel(x, token_table):
    pos = _positional_encoding()
    xf = x.reshape(BATCH * MAX_LEN)
    mesh = plsc.VectorSubcoreMesh(core_axis_name="c", subcore_axis_name="s")
    run = pl.kernel(
        _tpe_body,
        out_type=jax.ShapeDtypeStruct((BATCH * MAX_LEN, DIM), jnp.float32),
        mesh=mesh,
        scratch_types=[
            pltpu.VMEM((ROWS_W,), jnp.int32),
            pltpu.VMEM((MAX_LEN, DIM), jnp.float32),
            pltpu.VMEM((CHUNK, DIM), jnp.float32),
            pltpu.VMEM((CHUNK, DIM), jnp.float32),
            pltpu.SemaphoreType.DMA,
            pltpu.SemaphoreType.DMA,
            pltpu.SemaphoreType.DMA,
            pltpu.SemaphoreType.DMA,
        ],
        compiler_params=pltpu.CompilerParams(use_tc_tiling_on_sc=False),
    )
    outf = run(xf, pos, token_table)
    return outf.reshape(BATCH, MAX_LEN, DIM)
